# Initial kernel scaffold; baseline (speedup 1.0000x reference)
import functools

import jax
import jax.numpy as jnp
from jax import lax
from jax.experimental import pallas as pl
from jax.experimental.pallas import tpu as pltpu
from jax.experimental.pallas import tpu_sc as plsc

_N = 10000
_E = 320000
_D = 128
_H = 16
_G = 64
_CLS = 2

_NC = 2
_NS = 16
_L = 16

_N_PAD = 10016
_ROWS_PER_SUB = _N_PAD // _NS
_E_CHUNK = 128
_CHUNKS_PER_TILE = 79
_E_PAD = _NC * _NS * _CHUNKS_PER_TILE * _E_CHUNK
_E_ROWS = _E_PAD // _E_CHUNK

_BR = 2504
_TC_GRID = _N_PAD // _BR


def _sc_mesh():
    return plsc.VectorSubcoreMesh(core_axis_name="c", subcore_axis_name="s")


@functools.partial(
    pl.kernel,
    out_type=jax.ShapeDtypeStruct((_NC * _N_PAD, _L), jnp.float32),
    mesh=_sc_mesh(),
    scratch_types=[
        pltpu.VMEM((_CHUNKS_PER_TILE, _E_CHUNK), jnp.int32),
        pltpu.VMEM((_E_CHUNK, _L), jnp.float32),
        pltpu.VMEM_SHARED((_N_PAD, _L), jnp.float32),
    ],
)
def _deg_kernel(dst_hbm, ones_hbm, zeros_hbm, out_hbm, dst_v, ones_v, acc):
    c = lax.axis_index("c")
    s = lax.axis_index("s")
    gtile = c * _NS + s
    pltpu.sync_copy(dst_hbm.at[pl.ds(gtile * _CHUNKS_PER_TILE, _CHUNKS_PER_TILE)], dst_v)
    pltpu.sync_copy(ones_hbm, ones_v)
    pltpu.sync_copy(zeros_hbm, acc.at[pl.ds(s * _ROWS_PER_SUB, _ROWS_PER_SUB)])
    plsc.subcore_barrier()

    def body(j, carry):
        pltpu.sync_copy(ones_v, acc.at[dst_v.at[j]], add=True)
        return carry

    lax.fori_loop(0, _CHUNKS_PER_TILE, body, 0)
    plsc.subcore_barrier()
    pltpu.sync_copy(
        acc.at[pl.ds(s * _ROWS_PER_SUB, _ROWS_PER_SUB)],
        out_hbm.at[pl.ds(c * _N_PAD + s * _ROWS_PER_SUB, _ROWS_PER_SUB)],
    )


@functools.partial(
    pl.kernel,
    out_type=jax.ShapeDtypeStruct((_NC * _N_PAD, _L), jnp.float32),
    mesh=_sc_mesh(),
    scratch_types=[
        pltpu.VMEM((_CHUNKS_PER_TILE, _E_CHUNK), jnp.int32),
        pltpu.VMEM((_CHUNKS_PER_TILE, _E_CHUNK), jnp.int32),
        pltpu.VMEM((_E_CHUNK, _L), jnp.float32),
        pltpu.SemaphoreType.DMA,
        pltpu.VMEM_SHARED((_N_PAD, _L), jnp.float32),
    ],
)
def _edge_kernel(src_hbm, dst_hbm, rows_hbm, zeros_hbm, out_hbm,
                 src_v, dst_v, rows_v, sem, acc):
    c = lax.axis_index("c")
    s = lax.axis_index("s")
    gtile = c * _NS + s
    pltpu.sync_copy(src_hbm.at[pl.ds(gtile * _CHUNKS_PER_TILE, _CHUNKS_PER_TILE)], src_v)
    pltpu.sync_copy(dst_hbm.at[pl.ds(gtile * _CHUNKS_PER_TILE, _CHUNKS_PER_TILE)], dst_v)
    pltpu.sync_copy(zeros_hbm, acc.at[pl.ds(s * _ROWS_PER_SUB, _ROWS_PER_SUB)])
    plsc.subcore_barrier()

    def body(j, carry):
        pltpu.async_copy(rows_hbm.at[src_v.at[j]], rows_v, sem).wait()
        pltpu.sync_copy(rows_v, acc.at[dst_v.at[j]], add=True)
        return carry

    lax.fori_loop(0, _CHUNKS_PER_TILE, body, 0)
    plsc.subcore_barrier()
    pltpu.sync_copy(
        acc.at[pl.ds(s * _ROWS_PER_SUB, _ROWS_PER_SUB)],
        out_hbm.at[pl.ds(c * _N_PAD + s * _ROWS_PER_SUB, _ROWS_PER_SUB)],
    )


def _dense1_body(x_ref, w_ref, degp_ref, hn_ref, dis_ref):
    h = jnp.dot(x_ref[...], w_ref[...], preferred_element_type=jnp.float32)
    deg = degp_ref[0] + degp_ref[1] + 1.0
    dis = lax.rsqrt(deg)
    dis_ref[...] = dis
    hn_ref[...] = h * dis


def _dense1(x_p, W1, degp):
    return pl.pallas_call(
        _dense1_body,
        grid=(_TC_GRID,),
        in_specs=[
            pl.BlockSpec((_BR, _D), lambda i: (i, 0)),
            pl.BlockSpec((_D, _H), lambda i: (0, 0)),
            pl.BlockSpec((_NC, _BR, _L), lambda i: (0, i, 0)),
        ],
        out_specs=[
            pl.BlockSpec((_BR, _H), lambda i: (i, 0)),
            pl.BlockSpec((_BR, _L), lambda i: (i, 0)),
        ],
        out_shape=[
            jax.ShapeDtypeStruct((_N_PAD, _H), jnp.float32),
            jax.ShapeDtypeStruct((_N_PAD, _L), jnp.float32),
        ],
    )(x_p, W1, degp)


def _dense2_body(aggp_ref, hn1_ref, dis_ref, b1_ref, w2_ref, hn2_ref):
    pre = (aggp_ref[0] + aggp_ref[1] + hn1_ref[...]) * dis_ref[...] + b1_ref[...]
    a = jnp.maximum(pre, 0.0)
    h2 = jnp.dot(a, w2_ref[...], preferred_element_type=jnp.float32)
    hn2_ref[...] = h2 * dis_ref[...]


def _dense2(aggp, hn1, dis, b1, W2):
    return pl.pallas_call(
        _dense2_body,
        grid=(_TC_GRID,),
        in_specs=[
            pl.BlockSpec((_NC, _BR, _L), lambda i: (0, i, 0)),
            pl.BlockSpec((_BR, _H), lambda i: (i, 0)),
            pl.BlockSpec((_BR, _L), lambda i: (i, 0)),
            pl.BlockSpec((1, _H), lambda i: (0, 0)),
            pl.BlockSpec((_H, _H), lambda i: (0, 0)),
        ],
        out_specs=pl.BlockSpec((_BR, _H), lambda i: (i, 0)),
        out_shape=jax.ShapeDtypeStruct((_N_PAD, _H), jnp.float32),
    )(aggp, hn1, dis, b1, W2)


def _final_body(aggp_ref, hn2_ref, dis_ref, b2_ref, batch_ref, wc_ref, bc_ref,
                out_ref, psum, pcnt):
    i = pl.program_id(0)

    @pl.when(i == 0)
    def _init():
        psum[...] = jnp.zeros_like(psum)
        pcnt[...] = jnp.zeros_like(pcnt)

    out2 = (aggp_ref[0] + aggp_ref[1] + hn2_ref[...]) * dis_ref[...] + b2_ref[...]
    b = batch_ref[...]
    onehot = (b == lax.broadcasted_iota(jnp.int32, (1, _G), 1)).astype(jnp.float32)
    psum[...] += lax.dot_general(onehot, out2, (((0,), (0,)), ((), ())),
                                 preferred_element_type=jnp.float32)
    pcnt[...] += lax.dot_general(onehot, jnp.ones_like(out2), (((0,), (0,)), ((), ())),
                                 preferred_element_type=jnp.float32)

    @pl.when(i == _TC_GRID - 1)
    def _finish():
        pooled = psum[...] / jnp.maximum(pcnt[...], 1.0)
        logits = jnp.dot(pooled, wc_ref[...], preferred_element_type=jnp.float32)
        logits = logits + bc_ref[...]
        m = jnp.max(logits, axis=1, keepdims=True)
        e = jnp.exp(logits - m)
        lse = m + jnp.log(jnp.sum(e, axis=1, keepdims=True))
        out_ref[...] = logits - lse


def _final(aggp, hn2, dis, b2, batch_p, Wc, bc):
    return pl.pallas_call(
        _final_body,
        grid=(_TC_GRID,),
        in_specs=[
            pl.BlockSpec((_NC, _BR, _L), lambda i: (0, i, 0)),
            pl.BlockSpec((_BR, _H), lambda i: (i, 0)),
            pl.BlockSpec((_BR, _L), lambda i: (i, 0)),
            pl.BlockSpec((1, _H), lambda i: (0, 0)),
            pl.BlockSpec((_BR, 1), lambda i: (i, 0)),
            pl.BlockSpec((_H, _CLS), lambda i: (0, 0)),
            pl.BlockSpec((1, _CLS), lambda i: (0, 0)),
        ],
        out_specs=pl.BlockSpec((_G, _CLS), lambda i: (0, 0)),
        out_shape=jax.ShapeDtypeStruct((_G, _CLS), jnp.float32),
        scratch_shapes=[
            pltpu.VMEM((_G, _H), jnp.float32),
            pltpu.VMEM((_G, _H), jnp.float32),
        ],
    )(aggp, hn2, dis, b2, batch_p, Wc, bc)


def kernel(x, edge_index, batch, W1, b1, W2, b2, Wc, bc):
    src = edge_index[0]
    dst = edge_index[1]
    e_fill = jnp.full((_E_PAD - _E,), _N, dtype=jnp.int32)
    src2d = jnp.concatenate([src.astype(jnp.int32), e_fill]).reshape(_E_ROWS, _E_CHUNK)
    dst2d = jnp.concatenate([dst.astype(jnp.int32), e_fill]).reshape(_E_ROWS, _E_CHUNK)

    x_p = jnp.concatenate([x, jnp.zeros((_N_PAD - _N, _D), jnp.float32)])
    batch_p = jnp.concatenate(
        [batch.astype(jnp.int32), jnp.full((_N_PAD - _N,), _G, jnp.int32)]
    ).reshape(_N_PAD, 1)
    zeros = jnp.zeros((_ROWS_PER_SUB, _L), jnp.float32)
    ones = jnp.ones((_E_CHUNK, _L), jnp.float32)

    degp = _deg_kernel(dst2d, ones, zeros).reshape(_NC, _N_PAD, _L)
    hn1, dis = _dense1(x_p, W1, degp)
    agg1 = _edge_kernel(src2d, dst2d, hn1, zeros).reshape(_NC, _N_PAD, _L)
    hn2 = _dense2(agg1, hn1, dis, b1.reshape(1, _H), W2)
    agg2 = _edge_kernel(src2d, dst2d, hn2, zeros).reshape(_NC, _N_PAD, _L)
    return _final(agg2, hn2, dis, b2.reshape(1, _H), batch_p, Wc,
                  bc.reshape(1, _CLS))


# speedup vs baseline: 29.4535x; 29.4535x over previous
import functools

import jax
import jax.numpy as jnp
from jax import lax
from jax.experimental import pallas as pl
from jax.experimental.pallas import tpu as pltpu
from jax.experimental.pallas import tpu_sc as plsc

_N = 10000
_E = 320000
_D = 128
_H = 16
_G = 64
_CLS = 2

_NC = 2
_NS = 16
_L = 16

_N_PAD = 10112
_ROWS_PER_SUB = _N_PAD // _NS
_E_CHUNK = 128
_CHUNKS_PER_TILE = 80
_E_PAD = _NC * _NS * _CHUNKS_PER_TILE * _E_CHUNK
_E_ROWS = _E_PAD // _E_CHUNK

_BR = 2528
_TC_GRID = _N_PAD // _BR


def _sc_mesh():
    return plsc.VectorSubcoreMesh(core_axis_name="c", subcore_axis_name="s")


@functools.cache
def _make_deg_kernel():
    @functools.partial(
        pl.kernel,
        out_type=jax.ShapeDtypeStruct((_NC * _N_PAD, _L), jnp.float32),
        mesh=_sc_mesh(),
        scratch_types=[
            pltpu.VMEM((_CHUNKS_PER_TILE, _E_CHUNK), jnp.int32),
            pltpu.VMEM((_E_CHUNK, _L), jnp.float32),
            pltpu.VMEM_SHARED((_N_PAD, _L), jnp.float32),
        ],
        compiler_params=pltpu.CompilerParams(use_tc_tiling_on_sc=False),
    )
    def deg_kernel(dst_hbm, ones_hbm, zeros_hbm, out_hbm, dst_v, ones_v, acc):
        c = lax.axis_index("c")
        s = lax.axis_index("s")
        gtile = c * _NS + s
        pltpu.sync_copy(dst_hbm.at[pl.ds(gtile * _CHUNKS_PER_TILE, _CHUNKS_PER_TILE)], dst_v)
        pltpu.sync_copy(ones_hbm, ones_v)
        pltpu.sync_copy(zeros_hbm, acc.at[pl.ds(s * _ROWS_PER_SUB, _ROWS_PER_SUB)])
        plsc.subcore_barrier()

        def body(j, carry):
            pltpu.sync_copy(ones_v, acc.at[dst_v.at[j]], add=True)
            return carry

        lax.fori_loop(0, _CHUNKS_PER_TILE, body, 0)
        plsc.subcore_barrier()
        pltpu.sync_copy(
            acc.at[pl.ds(s * _ROWS_PER_SUB, _ROWS_PER_SUB)],
            out_hbm.at[pl.ds(c * _N_PAD + s * _ROWS_PER_SUB, _ROWS_PER_SUB)],
        )

    return deg_kernel


def _deg_kernel(dst2d, ones, zeros):
    return _make_deg_kernel()(dst2d, ones, zeros)


@functools.cache
def _make_edge_kernel():
    @functools.partial(
        pl.kernel,
        out_type=jax.ShapeDtypeStruct((_NC * _N_PAD, _L), jnp.float32),
        mesh=_sc_mesh(),
        scratch_types=[
            pltpu.VMEM((_CHUNKS_PER_TILE, _E_CHUNK), jnp.int32),
            pltpu.VMEM((_CHUNKS_PER_TILE, _E_CHUNK), jnp.int32),
            pltpu.VMEM((_E_CHUNK, _L), jnp.float32),
            pltpu.SemaphoreType.DMA,
            pltpu.VMEM_SHARED((_N_PAD, _L), jnp.float32),
        ],
        compiler_params=pltpu.CompilerParams(use_tc_tiling_on_sc=False),
    )
    def edge_kernel(src_hbm, dst_hbm, rows_hbm, zeros_hbm, out_hbm,
                    src_v, dst_v, rows_v, sem, acc):
        c = lax.axis_index("c")
        s = lax.axis_index("s")
        gtile = c * _NS + s
        pltpu.sync_copy(src_hbm.at[pl.ds(gtile * _CHUNKS_PER_TILE, _CHUNKS_PER_TILE)], src_v)
        pltpu.sync_copy(dst_hbm.at[pl.ds(gtile * _CHUNKS_PER_TILE, _CHUNKS_PER_TILE)], dst_v)
        pltpu.sync_copy(zeros_hbm, acc.at[pl.ds(s * _ROWS_PER_SUB, _ROWS_PER_SUB)])
        plsc.subcore_barrier()

        def body(j, carry):
            pltpu.async_copy(rows_hbm.at[src_v.at[j]], rows_v, sem).wait()
            pltpu.sync_copy(rows_v, acc.at[dst_v.at[j]], add=True)
            return carry

        lax.fori_loop(0, _CHUNKS_PER_TILE, body, 0)
        plsc.subcore_barrier()
        pltpu.sync_copy(
            acc.at[pl.ds(s * _ROWS_PER_SUB, _ROWS_PER_SUB)],
            out_hbm.at[pl.ds(c * _N_PAD + s * _ROWS_PER_SUB, _ROWS_PER_SUB)],
        )

    return edge_kernel


def _edge_kernel(src2d, dst2d, rows, zeros):
    return _make_edge_kernel()(src2d, dst2d, rows, zeros)


def _dense1_body(x_ref, w_ref, degp_ref, hn_ref, dis_ref):
    h = jnp.dot(x_ref[...], w_ref[...], preferred_element_type=jnp.float32)
    deg = degp_ref[0] + degp_ref[1] + 1.0
    dis = lax.rsqrt(deg)
    dis_ref[...] = dis
    hn_ref[...] = h * dis


def _dense1(x_p, W1, degp):
    return pl.pallas_call(
        _dense1_body,
        grid=(_TC_GRID,),
        in_specs=[
            pl.BlockSpec((_BR, _D), lambda i: (i, 0)),
            pl.BlockSpec((_D, _H), lambda i: (0, 0)),
            pl.BlockSpec((_NC, _BR, _L), lambda i: (0, i, 0)),
        ],
        out_specs=[
            pl.BlockSpec((_BR, _H), lambda i: (i, 0)),
            pl.BlockSpec((_BR, _L), lambda i: (i, 0)),
        ],
        out_shape=[
            jax.ShapeDtypeStruct((_N_PAD, _H), jnp.float32),
            jax.ShapeDtypeStruct((_N_PAD, _L), jnp.float32),
        ],
    )(x_p, W1, degp)


def _dense2_body(aggp_ref, hn1_ref, dis_ref, b1_ref, w2_ref, hn2_ref):
    pre = (aggp_ref[0] + aggp_ref[1] + hn1_ref[...]) * dis_ref[...] + b1_ref[...]
    a = jnp.maximum(pre, 0.0)
    h2 = jnp.dot(a, w2_ref[...], preferred_element_type=jnp.float32)
    hn2_ref[...] = h2 * dis_ref[...]


def _dense2(aggp, hn1, dis, b1, W2):
    return pl.pallas_call(
        _dense2_body,
        grid=(_TC_GRID,),
        in_specs=[
            pl.BlockSpec((_NC, _BR, _L), lambda i: (0, i, 0)),
            pl.BlockSpec((_BR, _H), lambda i: (i, 0)),
            pl.BlockSpec((_BR, _L), lambda i: (i, 0)),
            pl.BlockSpec((1, _H), lambda i: (0, 0)),
            pl.BlockSpec((_H, _H), lambda i: (0, 0)),
        ],
        out_specs=pl.BlockSpec((_BR, _H), lambda i: (i, 0)),
        out_shape=jax.ShapeDtypeStruct((_N_PAD, _H), jnp.float32),
    )(aggp, hn1, dis, b1, W2)


def _final_body(aggp_ref, hn2_ref, dis_ref, b2_ref, batch_ref, wc_ref, bc_ref,
                out_ref, psum, pcnt):
    i = pl.program_id(0)

    @pl.when(i == 0)
    def _init():
        psum[...] = jnp.zeros_like(psum)
        pcnt[...] = jnp.zeros_like(pcnt)

    out2 = (aggp_ref[0] + aggp_ref[1] + hn2_ref[...]) * dis_ref[...] + b2_ref[...]
    b = batch_ref[...]
    onehot = (b == lax.broadcasted_iota(jnp.int32, (1, _G), 1)).astype(jnp.float32)
    psum[...] += lax.dot_general(onehot, out2, (((0,), (0,)), ((), ())),
                                 preferred_element_type=jnp.float32)
    pcnt[...] += lax.dot_general(onehot, jnp.ones_like(out2), (((0,), (0,)), ((), ())),
                                 preferred_element_type=jnp.float32)

    @pl.when(i == _TC_GRID - 1)
    def _finish():
        pooled = psum[...] / jnp.maximum(pcnt[...], 1.0)
        logits = jnp.dot(pooled, wc_ref[...], preferred_element_type=jnp.float32)
        logits = logits + bc_ref[...]
        m = jnp.max(logits, axis=1, keepdims=True)
        e = jnp.exp(logits - m)
        lse = m + jnp.log(jnp.sum(e, axis=1, keepdims=True))
        out_ref[...] = logits - lse


def _final(aggp, hn2, dis, b2, batch_p, Wc, bc):
    return pl.pallas_call(
        _final_body,
        grid=(_TC_GRID,),
        in_specs=[
            pl.BlockSpec((_NC, _BR, _L), lambda i: (0, i, 0)),
            pl.BlockSpec((_BR, _H), lambda i: (i, 0)),
            pl.BlockSpec((_BR, _L), lambda i: (i, 0)),
            pl.BlockSpec((1, _H), lambda i: (0, 0)),
            pl.BlockSpec((_BR, 1), lambda i: (i, 0)),
            pl.BlockSpec((_H, _CLS), lambda i: (0, 0)),
            pl.BlockSpec((1, _CLS), lambda i: (0, 0)),
        ],
        out_specs=pl.BlockSpec((_G, _CLS), lambda i: (0, 0)),
        out_shape=jax.ShapeDtypeStruct((_G, _CLS), jnp.float32),
        scratch_shapes=[
            pltpu.VMEM((_G, _H), jnp.float32),
            pltpu.VMEM((_G, _H), jnp.float32),
        ],
    )(aggp, hn2, dis, b2, batch_p, Wc, bc)


def kernel(x, edge_index, batch, W1, b1, W2, b2, Wc, bc):
    src = edge_index[0]
    dst = edge_index[1]
    e_fill = jnp.full((_E_PAD - _E,), _N, dtype=jnp.int32)
    src2d = jnp.concatenate([src.astype(jnp.int32), e_fill]).reshape(_E_ROWS, _E_CHUNK)
    dst2d = jnp.concatenate([dst.astype(jnp.int32), e_fill]).reshape(_E_ROWS, _E_CHUNK)

    x_p = jnp.concatenate([x, jnp.zeros((_N_PAD - _N, _D), jnp.float32)])
    batch_p = jnp.concatenate(
        [batch.astype(jnp.int32), jnp.full((_N_PAD - _N,), _G, jnp.int32)]
    ).reshape(_N_PAD, 1)
    zeros = jnp.zeros((_ROWS_PER_SUB, _L), jnp.float32)
    ones = jnp.ones((_E_CHUNK, _L), jnp.float32)

    degp = _deg_kernel(dst2d, ones, zeros).reshape(_NC, _N_PAD, _L)
    hn1, dis = _dense1(x_p, W1, degp)
    agg1 = _edge_kernel(src2d, dst2d, hn1, zeros).reshape(_NC, _N_PAD, _L)
    hn2 = _dense2(agg1, hn1, dis, b1.reshape(1, _H), W2)
    agg2 = _edge_kernel(src2d, dst2d, hn2, zeros).reshape(_NC, _N_PAD, _L)
    return _final(agg2, hn2, dis, b2.reshape(1, _H), batch_p, Wc,
                  bc.reshape(1, _CLS))


# speedup vs baseline: 39.5807x; 1.3438x over previous
import functools

import jax
import jax.numpy as jnp
from jax import lax
from jax.experimental import pallas as pl
from jax.experimental.pallas import tpu as pltpu
from jax.experimental.pallas import tpu_sc as plsc

_N = 10000
_E = 320000
_D = 128
_H = 16
_G = 64
_CLS = 2

_NC = 2
_NS = 16
_L = 16

_N_PAD = 10112
_ROWS_PER_SUB = _N_PAD // _NS
_E_CHUNK = 128
_CHUNKS_PER_TILE = 80
_E_PAD = _NC * _NS * _CHUNKS_PER_TILE * _E_CHUNK
_E_ROWS = _E_PAD // _E_CHUNK

_BR = 2528
_TC_GRID = _N_PAD // _BR


def _sc_mesh():
    return plsc.VectorSubcoreMesh(core_axis_name="c", subcore_axis_name="s")


@functools.cache
def _make_deg_kernel():
    @functools.partial(
        pl.kernel,
        out_type=jax.ShapeDtypeStruct((_NC * _N_PAD, _L), jnp.float32),
        mesh=_sc_mesh(),
        scratch_types=[
            pltpu.VMEM((_CHUNKS_PER_TILE, _E_CHUNK), jnp.int32),
            pltpu.VMEM((_E_CHUNK, _L), jnp.float32),
            pltpu.VMEM_SHARED((_N_PAD, _L), jnp.float32),
        ],
        compiler_params=pltpu.CompilerParams(use_tc_tiling_on_sc=False),
    )
    def deg_kernel(dst_hbm, ones_hbm, zeros_hbm, out_hbm, dst_v, ones_v, acc):
        c = lax.axis_index("c")
        s = lax.axis_index("s")
        gtile = c * _NS + s
        pltpu.sync_copy(dst_hbm.at[pl.ds(gtile * _CHUNKS_PER_TILE, _CHUNKS_PER_TILE)], dst_v)
        pltpu.sync_copy(ones_hbm, ones_v)
        pltpu.sync_copy(zeros_hbm, acc.at[pl.ds(s * _ROWS_PER_SUB, _ROWS_PER_SUB)])
        plsc.subcore_barrier()

        def body(j, carry):
            pltpu.sync_copy(ones_v, acc.at[dst_v.at[j]], add=True)
            return carry

        lax.fori_loop(0, _CHUNKS_PER_TILE, body, 0)
        plsc.subcore_barrier()
        pltpu.sync_copy(
            acc.at[pl.ds(s * _ROWS_PER_SUB, _ROWS_PER_SUB)],
            out_hbm.at[pl.ds(c * _N_PAD + s * _ROWS_PER_SUB, _ROWS_PER_SUB)],
        )

    return deg_kernel


def _deg_kernel(dst2d, ones, zeros):
    return _make_deg_kernel()(dst2d, ones, zeros)


_NBUF = 4


@functools.cache
def _make_edge_kernel():
    @functools.partial(
        pl.kernel,
        out_type=jax.ShapeDtypeStruct((_NC * _N_PAD, _L), jnp.float32),
        mesh=_sc_mesh(),
        scratch_types=[
            pltpu.VMEM((_CHUNKS_PER_TILE, _E_CHUNK), jnp.int32),
            pltpu.VMEM((_CHUNKS_PER_TILE, _E_CHUNK), jnp.int32),
            pltpu.VMEM((_NBUF, _E_CHUNK, _L), jnp.float32),
        ]
        + [pltpu.SemaphoreType.DMA] * _NBUF
        + [pltpu.VMEM_SHARED((_N_PAD, _L), jnp.float32)],
        compiler_params=pltpu.CompilerParams(use_tc_tiling_on_sc=False),
    )
    def edge_kernel(src_hbm, dst_hbm, rows_hbm, zeros_hbm, out_hbm,
                    src_v, dst_v, rows_v, sem0, sem1, sem2, sem3, acc):
        sems = (sem0, sem1, sem2, sem3)
        c = lax.axis_index("c")
        s = lax.axis_index("s")
        gtile = c * _NS + s
        pltpu.sync_copy(src_hbm.at[pl.ds(gtile * _CHUNKS_PER_TILE, _CHUNKS_PER_TILE)], src_v)
        pltpu.sync_copy(dst_hbm.at[pl.ds(gtile * _CHUNKS_PER_TILE, _CHUNKS_PER_TILE)], dst_v)
        pltpu.sync_copy(zeros_hbm, acc.at[pl.ds(s * _ROWS_PER_SUB, _ROWS_PER_SUB)])
        plsc.subcore_barrier()

        for b in range(_NBUF):
            pltpu.async_copy(rows_hbm.at[src_v.at[b]], rows_v.at[b], sems[b])

        def body(g, carry):
            for b in range(_NBUF):
                j = g * _NBUF + b
                pltpu.make_async_copy(
                    rows_hbm.at[src_v.at[j]], rows_v.at[b], sems[b]
                ).wait()
                pltpu.sync_copy(rows_v.at[b], acc.at[dst_v.at[j]], add=True)
                nxt = j + _NBUF

                @pl.when(nxt < _CHUNKS_PER_TILE)
                def _refill():
                    pltpu.async_copy(rows_hbm.at[src_v.at[nxt]], rows_v.at[b], sems[b])

            return carry

        lax.fori_loop(0, _CHUNKS_PER_TILE // _NBUF, body, 0)
        plsc.subcore_barrier()
        pltpu.sync_copy(
            acc.at[pl.ds(s * _ROWS_PER_SUB, _ROWS_PER_SUB)],
            out_hbm.at[pl.ds(c * _N_PAD + s * _ROWS_PER_SUB, _ROWS_PER_SUB)],
        )

    return edge_kernel


def _edge_kernel(src2d, dst2d, rows, zeros):
    return _make_edge_kernel()(src2d, dst2d, rows, zeros)


def _dense1_body(x_ref, w_ref, degp_ref, hn_ref, dis_ref):
    h = jnp.dot(x_ref[...], w_ref[...], preferred_element_type=jnp.float32)
    deg = degp_ref[0] + degp_ref[1] + 1.0
    dis = lax.rsqrt(deg)
    dis_ref[...] = dis
    hn_ref[...] = h * dis


def _dense1(x_p, W1, degp):
    return pl.pallas_call(
        _dense1_body,
        grid=(_TC_GRID,),
        in_specs=[
            pl.BlockSpec((_BR, _D), lambda i: (i, 0)),
            pl.BlockSpec((_D, _H), lambda i: (0, 0)),
            pl.BlockSpec((_NC, _BR, _L), lambda i: (0, i, 0)),
        ],
        out_specs=[
            pl.BlockSpec((_BR, _H), lambda i: (i, 0)),
            pl.BlockSpec((_BR, _L), lambda i: (i, 0)),
        ],
        out_shape=[
            jax.ShapeDtypeStruct((_N_PAD, _H), jnp.float32),
            jax.ShapeDtypeStruct((_N_PAD, _L), jnp.float32),
        ],
    )(x_p, W1, degp)


def _dense2_body(aggp_ref, hn1_ref, dis_ref, b1_ref, w2_ref, hn2_ref):
    pre = (aggp_ref[0] + aggp_ref[1] + hn1_ref[...]) * dis_ref[...] + b1_ref[...]
    a = jnp.maximum(pre, 0.0)
    h2 = jnp.dot(a, w2_ref[...], preferred_element_type=jnp.float32)
    hn2_ref[...] = h2 * dis_ref[...]


def _dense2(aggp, hn1, dis, b1, W2):
    return pl.pallas_call(
        _dense2_body,
        grid=(_TC_GRID,),
        in_specs=[
            pl.BlockSpec((_NC, _BR, _L), lambda i: (0, i, 0)),
            pl.BlockSpec((_BR, _H), lambda i: (i, 0)),
            pl.BlockSpec((_BR, _L), lambda i: (i, 0)),
            pl.BlockSpec((1, _H), lambda i: (0, 0)),
            pl.BlockSpec((_H, _H), lambda i: (0, 0)),
        ],
        out_specs=pl.BlockSpec((_BR, _H), lambda i: (i, 0)),
        out_shape=jax.ShapeDtypeStruct((_N_PAD, _H), jnp.float32),
    )(aggp, hn1, dis, b1, W2)


def _final_body(aggp_ref, hn2_ref, dis_ref, b2_ref, batch_ref, wc_ref, bc_ref,
                out_ref, psum, pcnt):
    i = pl.program_id(0)

    @pl.when(i == 0)
    def _init():
        psum[...] = jnp.zeros_like(psum)
        pcnt[...] = jnp.zeros_like(pcnt)

    out2 = (aggp_ref[0] + aggp_ref[1] + hn2_ref[...]) * dis_ref[...] + b2_ref[...]
    b = batch_ref[...]
    onehot = (b == lax.broadcasted_iota(jnp.int32, (1, _G), 1)).astype(jnp.float32)
    psum[...] += lax.dot_general(onehot, out2, (((0,), (0,)), ((), ())),
                                 preferred_element_type=jnp.float32)
    pcnt[...] += lax.dot_general(onehot, jnp.ones_like(out2), (((0,), (0,)), ((), ())),
                                 preferred_element_type=jnp.float32)

    @pl.when(i == _TC_GRID - 1)
    def _finish():
        pooled = psum[...] / jnp.maximum(pcnt[...], 1.0)
        logits = jnp.dot(pooled, wc_ref[...], preferred_element_type=jnp.float32)
        logits = logits + bc_ref[...]
        m = jnp.max(logits, axis=1, keepdims=True)
        e = jnp.exp(logits - m)
        lse = m + jnp.log(jnp.sum(e, axis=1, keepdims=True))
        out_ref[...] = logits - lse


def _final(aggp, hn2, dis, b2, batch_p, Wc, bc):
    return pl.pallas_call(
        _final_body,
        grid=(_TC_GRID,),
        in_specs=[
            pl.BlockSpec((_NC, _BR, _L), lambda i: (0, i, 0)),
            pl.BlockSpec((_BR, _H), lambda i: (i, 0)),
            pl.BlockSpec((_BR, _L), lambda i: (i, 0)),
            pl.BlockSpec((1, _H), lambda i: (0, 0)),
            pl.BlockSpec((_BR, 1), lambda i: (i, 0)),
            pl.BlockSpec((_H, _CLS), lambda i: (0, 0)),
            pl.BlockSpec((1, _CLS), lambda i: (0, 0)),
        ],
        out_specs=pl.BlockSpec((_G, _CLS), lambda i: (0, 0)),
        out_shape=jax.ShapeDtypeStruct((_G, _CLS), jnp.float32),
        scratch_shapes=[
            pltpu.VMEM((_G, _H), jnp.float32),
            pltpu.VMEM((_G, _H), jnp.float32),
        ],
    )(aggp, hn2, dis, b2, batch_p, Wc, bc)


def kernel(x, edge_index, batch, W1, b1, W2, b2, Wc, bc):
    src = edge_index[0]
    dst = edge_index[1]
    e_fill = jnp.full((_E_PAD - _E,), _N, dtype=jnp.int32)
    src2d = jnp.concatenate([src.astype(jnp.int32), e_fill]).reshape(_E_ROWS, _E_CHUNK)
    dst2d = jnp.concatenate([dst.astype(jnp.int32), e_fill]).reshape(_E_ROWS, _E_CHUNK)

    x_p = jnp.concatenate([x, jnp.zeros((_N_PAD - _N, _D), jnp.float32)])
    batch_p = jnp.concatenate(
        [batch.astype(jnp.int32), jnp.full((_N_PAD - _N,), _G, jnp.int32)]
    ).reshape(_N_PAD, 1)
    zeros = jnp.zeros((_ROWS_PER_SUB, _L), jnp.float32)
    ones = jnp.ones((_E_CHUNK, _L), jnp.float32)

    degp = _deg_kernel(dst2d, ones, zeros).reshape(_NC, _N_PAD, _L)
    hn1, dis = _dense1(x_p, W1, degp)
    agg1 = _edge_kernel(src2d, dst2d, hn1, zeros).reshape(_NC, _N_PAD, _L)
    hn2 = _dense2(agg1, hn1, dis, b1.reshape(1, _H), W2)
    agg2 = _edge_kernel(src2d, dst2d, hn2, zeros).reshape(_NC, _N_PAD, _L)
    return _final(agg2, hn2, dis, b2.reshape(1, _H), batch_p, Wc,
                  bc.reshape(1, _CLS))


# speedup vs baseline: 58.0526x; 1.4667x over previous
import functools

import jax
import jax.numpy as jnp
from jax import lax
from jax.experimental import pallas as pl
from jax.experimental.pallas import tpu as pltpu
from jax.experimental.pallas import tpu_sc as plsc

_N = 10000
_E = 320000
_D = 128
_H = 16
_G = 64
_CLS = 2

_NC = 2
_NS = 16
_L = 16

_N_PAD = 10112
_ROWS_PER_SUB = _N_PAD // _NS
_E_CHUNK = 128
_CHUNKS_PER_TILE = 80
_E_PAD = _NC * _NS * _CHUNKS_PER_TILE * _E_CHUNK
_E_ROWS = _E_PAD // _E_CHUNK

_BR = 2528
_TC_GRID = _N_PAD // _BR


def _sc_mesh():
    return plsc.VectorSubcoreMesh(core_axis_name="c", subcore_axis_name="s")


@functools.cache
def _make_deg_kernel():
    @functools.partial(
        pl.kernel,
        out_type=jax.ShapeDtypeStruct((_NC * _N_PAD, _L), jnp.float32),
        mesh=_sc_mesh(),
        scratch_types=[
            pltpu.VMEM((_CHUNKS_PER_TILE, _E_CHUNK), jnp.int32),
            pltpu.VMEM((_E_CHUNK, _L), jnp.float32),
            pltpu.VMEM_SHARED((_N_PAD, _L), jnp.float32),
        ],
        compiler_params=pltpu.CompilerParams(use_tc_tiling_on_sc=False),
    )
    def deg_kernel(dst_hbm, ones_hbm, zeros_hbm, out_hbm, dst_v, ones_v, acc):
        c = lax.axis_index("c")
        s = lax.axis_index("s")
        gtile = c * _NS + s
        pltpu.sync_copy(dst_hbm.at[pl.ds(gtile * _CHUNKS_PER_TILE, _CHUNKS_PER_TILE)], dst_v)
        pltpu.sync_copy(ones_hbm, ones_v)
        pltpu.sync_copy(zeros_hbm, acc.at[pl.ds(s * _ROWS_PER_SUB, _ROWS_PER_SUB)])
        plsc.subcore_barrier()

        def body(j, carry):
            pltpu.sync_copy(ones_v, acc.at[dst_v.at[j]], add=True)
            return carry

        lax.fori_loop(0, _CHUNKS_PER_TILE, body, 0)
        plsc.subcore_barrier()
        pltpu.sync_copy(
            acc.at[pl.ds(s * _ROWS_PER_SUB, _ROWS_PER_SUB)],
            out_hbm.at[pl.ds(c * _N_PAD + s * _ROWS_PER_SUB, _ROWS_PER_SUB)],
        )

    return deg_kernel


def _deg_kernel(dst2d, ones, zeros):
    return _make_deg_kernel()(dst2d, ones, zeros)


_NBUF = 4


@functools.cache
def _make_edge_kernel():
    @functools.partial(
        pl.kernel,
        out_type=jax.ShapeDtypeStruct((_NC * _N_PAD, _L), jnp.float32),
        mesh=_sc_mesh(),
        scratch_types=[
            pltpu.VMEM((_CHUNKS_PER_TILE, _E_CHUNK), jnp.int32),
            pltpu.VMEM((_CHUNKS_PER_TILE, _E_CHUNK), jnp.int32),
            pltpu.VMEM((_NBUF, _E_CHUNK, _L), jnp.float32),
        ]
        + [pltpu.SemaphoreType.DMA] * _NBUF
        + [
            pltpu.VMEM_SHARED((_N_PAD, _L), jnp.float32),
            pltpu.VMEM_SHARED((_N_PAD, _L), jnp.float32),
        ],
        compiler_params=pltpu.CompilerParams(use_tc_tiling_on_sc=False),
    )
    def edge_kernel(src_hbm, dst_hbm, rows_hbm, zeros_hbm, out_hbm,
                    src_v, dst_v, rows_v, sem0, sem1, sem2, sem3, acc, hn_s):
        sems = (sem0, sem1, sem2, sem3)
        c = lax.axis_index("c")
        s = lax.axis_index("s")
        gtile = c * _NS + s
        pltpu.sync_copy(src_hbm.at[pl.ds(gtile * _CHUNKS_PER_TILE, _CHUNKS_PER_TILE)], src_v)
        pltpu.sync_copy(dst_hbm.at[pl.ds(gtile * _CHUNKS_PER_TILE, _CHUNKS_PER_TILE)], dst_v)
        pltpu.sync_copy(zeros_hbm, acc.at[pl.ds(s * _ROWS_PER_SUB, _ROWS_PER_SUB)])
        pltpu.sync_copy(
            rows_hbm.at[pl.ds(s * _ROWS_PER_SUB, _ROWS_PER_SUB)],
            hn_s.at[pl.ds(s * _ROWS_PER_SUB, _ROWS_PER_SUB)],
        )
        plsc.subcore_barrier()

        for b in range(_NBUF):
            pltpu.async_copy(hn_s.at[src_v.at[b]], rows_v.at[b], sems[b])

        def body(g, carry):
            for b in range(_NBUF):
                j = g * _NBUF + b
                pltpu.make_async_copy(
                    hn_s.at[src_v.at[j]], rows_v.at[b], sems[b]
                ).wait()
                pltpu.sync_copy(rows_v.at[b], acc.at[dst_v.at[j]], add=True)
                nxt = j + _NBUF

                @pl.when(nxt < _CHUNKS_PER_TILE)
                def _refill():
                    pltpu.async_copy(hn_s.at[src_v.at[nxt]], rows_v.at[b], sems[b])

            return carry

        lax.fori_loop(0, _CHUNKS_PER_TILE // _NBUF, body, 0)
        plsc.subcore_barrier()
        pltpu.sync_copy(
            acc.at[pl.ds(s * _ROWS_PER_SUB, _ROWS_PER_SUB)],
            out_hbm.at[pl.ds(c * _N_PAD + s * _ROWS_PER_SUB, _ROWS_PER_SUB)],
        )

    return edge_kernel


def _edge_kernel(src2d, dst2d, rows, zeros):
    return _make_edge_kernel()(src2d, dst2d, rows, zeros)


def _dense1_body(x_ref, w_ref, degp_ref, hn_ref, dis_ref):
    h = jnp.dot(x_ref[...], w_ref[...], preferred_element_type=jnp.float32)
    deg = degp_ref[0] + degp_ref[1] + 1.0
    dis = lax.rsqrt(deg)
    dis_ref[...] = dis
    hn_ref[...] = h * dis


def _dense1(x_p, W1, degp):
    return pl.pallas_call(
        _dense1_body,
        grid=(_TC_GRID,),
        in_specs=[
            pl.BlockSpec((_BR, _D), lambda i: (i, 0)),
            pl.BlockSpec((_D, _H), lambda i: (0, 0)),
            pl.BlockSpec((_NC, _BR, _L), lambda i: (0, i, 0)),
        ],
        out_specs=[
            pl.BlockSpec((_BR, _H), lambda i: (i, 0)),
            pl.BlockSpec((_BR, _L), lambda i: (i, 0)),
        ],
        out_shape=[
            jax.ShapeDtypeStruct((_N_PAD, _H), jnp.float32),
            jax.ShapeDtypeStruct((_N_PAD, _L), jnp.float32),
        ],
    )(x_p, W1, degp)


def _dense2_body(aggp_ref, hn1_ref, dis_ref, b1_ref, w2_ref, hn2_ref):
    pre = (aggp_ref[0] + aggp_ref[1] + hn1_ref[...]) * dis_ref[...] + b1_ref[...]
    a = jnp.maximum(pre, 0.0)
    h2 = jnp.dot(a, w2_ref[...], preferred_element_type=jnp.float32)
    hn2_ref[...] = h2 * dis_ref[...]


def _dense2(aggp, hn1, dis, b1, W2):
    return pl.pallas_call(
        _dense2_body,
        grid=(_TC_GRID,),
        in_specs=[
            pl.BlockSpec((_NC, _BR, _L), lambda i: (0, i, 0)),
            pl.BlockSpec((_BR, _H), lambda i: (i, 0)),
            pl.BlockSpec((_BR, _L), lambda i: (i, 0)),
            pl.BlockSpec((1, _H), lambda i: (0, 0)),
            pl.BlockSpec((_H, _H), lambda i: (0, 0)),
        ],
        out_specs=pl.BlockSpec((_BR, _H), lambda i: (i, 0)),
        out_shape=jax.ShapeDtypeStruct((_N_PAD, _H), jnp.float32),
    )(aggp, hn1, dis, b1, W2)


def _final_body(aggp_ref, hn2_ref, dis_ref, b2_ref, batch_ref, wc_ref, bc_ref,
                out_ref, psum, pcnt):
    i = pl.program_id(0)

    @pl.when(i == 0)
    def _init():
        psum[...] = jnp.zeros_like(psum)
        pcnt[...] = jnp.zeros_like(pcnt)

    out2 = (aggp_ref[0] + aggp_ref[1] + hn2_ref[...]) * dis_ref[...] + b2_ref[...]
    b = batch_ref[...]
    onehot = (b == lax.broadcasted_iota(jnp.int32, (1, _G), 1)).astype(jnp.float32)
    psum[...] += lax.dot_general(onehot, out2, (((0,), (0,)), ((), ())),
                                 preferred_element_type=jnp.float32)
    pcnt[...] += lax.dot_general(onehot, jnp.ones_like(out2), (((0,), (0,)), ((), ())),
                                 preferred_element_type=jnp.float32)

    @pl.when(i == _TC_GRID - 1)
    def _finish():
        pooled = psum[...] / jnp.maximum(pcnt[...], 1.0)
        logits = jnp.dot(pooled, wc_ref[...], preferred_element_type=jnp.float32)
        logits = logits + bc_ref[...]
        m = jnp.max(logits, axis=1, keepdims=True)
        e = jnp.exp(logits - m)
        lse = m + jnp.log(jnp.sum(e, axis=1, keepdims=True))
        out_ref[...] = logits - lse


def _final(aggp, hn2, dis, b2, batch_p, Wc, bc):
    return pl.pallas_call(
        _final_body,
        grid=(_TC_GRID,),
        in_specs=[
            pl.BlockSpec((_NC, _BR, _L), lambda i: (0, i, 0)),
            pl.BlockSpec((_BR, _H), lambda i: (i, 0)),
            pl.BlockSpec((_BR, _L), lambda i: (i, 0)),
            pl.BlockSpec((1, _H), lambda i: (0, 0)),
            pl.BlockSpec((_BR, 1), lambda i: (i, 0)),
            pl.BlockSpec((_H, _CLS), lambda i: (0, 0)),
            pl.BlockSpec((1, _CLS), lambda i: (0, 0)),
        ],
        out_specs=pl.BlockSpec((_G, _CLS), lambda i: (0, 0)),
        out_shape=jax.ShapeDtypeStruct((_G, _CLS), jnp.float32),
        scratch_shapes=[
            pltpu.VMEM((_G, _H), jnp.float32),
            pltpu.VMEM((_G, _H), jnp.float32),
        ],
    )(aggp, hn2, dis, b2, batch_p, Wc, bc)


def kernel(x, edge_index, batch, W1, b1, W2, b2, Wc, bc):
    src = edge_index[0]
    dst = edge_index[1]
    e_fill = jnp.full((_E_PAD - _E,), _N, dtype=jnp.int32)
    src2d = jnp.concatenate([src.astype(jnp.int32), e_fill]).reshape(_E_ROWS, _E_CHUNK)
    dst2d = jnp.concatenate([dst.astype(jnp.int32), e_fill]).reshape(_E_ROWS, _E_CHUNK)

    x_p = jnp.concatenate([x, jnp.zeros((_N_PAD - _N, _D), jnp.float32)])
    batch_p = jnp.concatenate(
        [batch.astype(jnp.int32), jnp.full((_N_PAD - _N,), _G, jnp.int32)]
    ).reshape(_N_PAD, 1)
    zeros = jnp.zeros((_ROWS_PER_SUB, _L), jnp.float32)
    ones = jnp.ones((_E_CHUNK, _L), jnp.float32)

    degp = _deg_kernel(dst2d, ones, zeros).reshape(_NC, _N_PAD, _L)
    hn1, dis = _dense1(x_p, W1, degp)
    agg1 = _edge_kernel(src2d, dst2d, hn1, zeros).reshape(_NC, _N_PAD, _L)
    hn2 = _dense2(agg1, hn1, dis, b1.reshape(1, _H), W2)
    agg2 = _edge_kernel(src2d, dst2d, hn2, zeros).reshape(_NC, _N_PAD, _L)
    return _final(agg2, hn2, dis, b2.reshape(1, _H), batch_p, Wc,
                  bc.reshape(1, _CLS))
